# Initial kernel scaffold; baseline (speedup 1.0000x reference)
import functools

import jax
import jax.numpy as jnp
from jax import lax
from jax.experimental import pallas as pl
from jax.experimental.pallas import tpu as pltpu
from jax.experimental.pallas import tpu_sc as plsc

NC, NS = 2, 16
NW = NC * NS
LANES = 16
HALF = 100
BLKS = 8


def _make_pool_kernel(B, S, V, D):
    assert D == 2 * LANES
    assert S == 2 * HALF
    assert B % NW == 0
    rw = B // NW
    rpb = rw // BLKS
    assert rpb % 2 == 0
    inv = 1.0 / S

    mesh = plsc.VectorSubcoreMesh(core_axis_name="c", subcore_axis_name="s")

    @functools.partial(
        pl.kernel,
        out_type=jax.ShapeDtypeStruct((B, D), jnp.float32),
        mesh=mesh,
        scratch_types=[
            pltpu.VMEM((2 * rpb, HALF), jnp.int32),
            pltpu.VMEM((2, 2, HALF, D), jnp.float32),
            pltpu.VMEM((rpb, D), jnp.float32),
            pltpu.SemaphoreType.DMA,
            pltpu.SemaphoreType.DMA,
            pltpu.SemaphoreType.DMA,
            pltpu.SemaphoreType.DMA,
        ],
    )
    def pool(idx_hbm, table_hbm, out_hbm, idx_v, rows_v, out_v,
             sem00, sem01, sem10, sem11):
        wid = lax.axis_index("s") * NC + lax.axis_index("c")
        sems = ((sem00, sem01), (sem10, sem11))

        def start_row(j, p):
            pltpu.async_copy(table_hbm.at[idx_v.at[2 * j]],
                             rows_v.at[p, 0], sems[p][0])
            pltpu.async_copy(table_hbm.at[idx_v.at[2 * j + 1]],
                             rows_v.at[p, 1], sems[p][1])

        def wait_pair(p):
            pltpu.make_async_copy(table_hbm.at[idx_v.at[0]],
                                  rows_v.at[p, 0], sems[p][0]).wait()
            pltpu.make_async_copy(table_hbm.at[idx_v.at[1]],
                                  rows_v.at[p, 1], sems[p][1]).wait()

        def reduce_pair(p, r):
            z = jnp.zeros((LANES,), jnp.float32)

            def body(i, accs):
                a0, a1, b0, b1 = accs
                a0 = a0 + rows_v[p, 0, i, pl.ds(0, LANES)]
                a1 = a1 + rows_v[p, 0, i, pl.ds(LANES, LANES)]
                b0 = b0 + rows_v[p, 1, i, pl.ds(0, LANES)]
                b1 = b1 + rows_v[p, 1, i, pl.ds(LANES, LANES)]
                return a0, a1, b0, b1

            a0, a1, b0, b1 = lax.fori_loop(0, HALF, body, (z, z, z, z))
            out_v[r, pl.ds(0, LANES)] = (a0 + b0) * inv
            out_v[r, pl.ds(LANES, LANES)] = (a1 + b1) * inv

        def blk_body(blk, _):
            pltpu.sync_copy(idx_hbm.at[wid, pl.ds(blk * 2 * rpb, 2 * rpb)],
                            idx_v)
            start_row(0, 0)

            def rp_body(rp, _):
                r = 2 * rp
                start_row(r + 1, 1)
                wait_pair(0)
                reduce_pair(0, r)

                @pl.when(rp < rpb // 2 - 1)
                def _():
                    start_row(r + 2, 0)

                wait_pair(1)
                reduce_pair(1, r + 1)
                return 0

            lax.fori_loop(0, rpb // 2, rp_body, 0)
            pltpu.sync_copy(out_v,
                            out_hbm.at[pl.ds(wid * rw + blk * rpb, rpb)])
            return 0

        lax.fori_loop(0, BLKS, blk_body, 0)

    return pool


def _matmul_bias(pooled, W, b):
    B, D = pooled.shape
    L = W.shape[0]
    BB = 2048

    def mm_body(x_ref, w_ref, b_ref, o_ref):
        o_ref[...] = (
            jnp.dot(x_ref[...], w_ref[...].T,
                    preferred_element_type=jnp.float32)
            + b_ref[...]
        )

    return pl.pallas_call(
        mm_body,
        out_shape=jax.ShapeDtypeStruct((B, L), jnp.float32),
        grid=(B // BB,),
        in_specs=[
            pl.BlockSpec((BB, D), lambda i: (i, 0)),
            pl.BlockSpec((L, D), lambda i: (0, 0)),
            pl.BlockSpec((1, L), lambda i: (0, 0)),
        ],
        out_specs=pl.BlockSpec((BB, L), lambda i: (i, 0)),
    )(pooled, W, b.reshape(1, L))


@jax.jit
def kernel(inputs, emb_table, W, b):
    B, S = inputs.shape
    V, D = emb_table.shape
    idx3 = inputs.astype(jnp.int32).reshape(NW, (B // NW) * 2, HALF)
    pooled = _make_pool_kernel(B, S, V, D)(idx3, emb_table)
    return _matmul_bias(pooled, W, b)


# speedup vs baseline: 13.5853x; 13.5853x over previous
import functools

import jax
import jax.numpy as jnp
from jax import lax
from jax.experimental import pallas as pl
from jax.experimental.pallas import tpu as pltpu
from jax.experimental.pallas import tpu_sc as plsc

NC, NS = 2, 16
NW = NC * NS
LANES = 16
HALF = 100
BLKS = 8


def _make_pool_kernel(B, S, V, D):
    assert D == 2 * LANES
    assert S == 2 * HALF
    assert B % NW == 0
    rw = B // NW
    rpb = rw // BLKS
    assert rpb % 2 == 0
    inv = 1.0 / S

    mesh = plsc.VectorSubcoreMesh(core_axis_name="c", subcore_axis_name="s")

    @functools.partial(
        pl.kernel,
        out_type=jax.ShapeDtypeStruct((B, D), jnp.float32),
        mesh=mesh,
        compiler_params=pltpu.CompilerParams(use_tc_tiling_on_sc=False),
        scratch_types=[
            pltpu.VMEM((2 * rpb, HALF), jnp.int32),
            pltpu.VMEM((2, 2, HALF, D), jnp.float32),
            pltpu.VMEM((rpb, D), jnp.float32),
            pltpu.SemaphoreType.DMA,
            pltpu.SemaphoreType.DMA,
            pltpu.SemaphoreType.DMA,
            pltpu.SemaphoreType.DMA,
        ],
    )
    def pool(idx_hbm, table_hbm, out_hbm, idx_v, rows_v, out_v,
             sem00, sem01, sem10, sem11):
        wid = lax.axis_index("s") * NC + lax.axis_index("c")
        sems = ((sem00, sem01), (sem10, sem11))

        def start_row(j, p):
            pltpu.async_copy(table_hbm.at[idx_v.at[2 * j]],
                             rows_v.at[p, 0], sems[p][0])
            pltpu.async_copy(table_hbm.at[idx_v.at[2 * j + 1]],
                             rows_v.at[p, 1], sems[p][1])

        def wait_pair(p):
            pltpu.make_async_copy(table_hbm.at[idx_v.at[0]],
                                  rows_v.at[p, 0], sems[p][0]).wait()
            pltpu.make_async_copy(table_hbm.at[idx_v.at[1]],
                                  rows_v.at[p, 1], sems[p][1]).wait()

        def reduce_pair(p, r):
            z = jnp.zeros((LANES,), jnp.float32)

            def body(i, accs):
                a0, a1, b0, b1 = accs
                a0 = a0 + rows_v[p, 0, i, pl.ds(0, LANES)]
                a1 = a1 + rows_v[p, 0, i, pl.ds(LANES, LANES)]
                b0 = b0 + rows_v[p, 1, i, pl.ds(0, LANES)]
                b1 = b1 + rows_v[p, 1, i, pl.ds(LANES, LANES)]
                return a0, a1, b0, b1

            a0, a1, b0, b1 = lax.fori_loop(0, HALF, body, (z, z, z, z))
            out_v[r, pl.ds(0, LANES)] = (a0 + b0) * inv
            out_v[r, pl.ds(LANES, LANES)] = (a1 + b1) * inv

        def blk_body(blk, _):
            pltpu.sync_copy(idx_hbm.at[wid, pl.ds(blk * 2 * rpb, 2 * rpb)],
                            idx_v)
            start_row(0, 0)

            def rp_body(rp, _):
                r = 2 * rp
                start_row(r + 1, 1)
                wait_pair(0)
                reduce_pair(0, r)

                @pl.when(rp < rpb // 2 - 1)
                def _():
                    start_row(r + 2, 0)

                wait_pair(1)
                reduce_pair(1, r + 1)
                return 0

            lax.fori_loop(0, rpb // 2, rp_body, 0)
            pltpu.sync_copy(out_v,
                            out_hbm.at[pl.ds(wid * rw + blk * rpb, rpb)])
            return 0

        lax.fori_loop(0, BLKS, blk_body, 0)

    return pool


def _matmul_bias(pooled, W, b):
    B, D = pooled.shape
    L = W.shape[0]
    BB = 2048

    def mm_body(x_ref, w_ref, b_ref, o_ref):
        o_ref[...] = (
            jnp.dot(x_ref[...], w_ref[...].T,
                    preferred_element_type=jnp.float32)
            + b_ref[...]
        )

    return pl.pallas_call(
        mm_body,
        out_shape=jax.ShapeDtypeStruct((B, L), jnp.float32),
        grid=(B // BB,),
        in_specs=[
            pl.BlockSpec((BB, D), lambda i: (i, 0)),
            pl.BlockSpec((L, D), lambda i: (0, 0)),
            pl.BlockSpec((1, L), lambda i: (0, 0)),
        ],
        out_specs=pl.BlockSpec((BB, L), lambda i: (i, 0)),
    )(pooled, W, b.reshape(1, L))


@jax.jit
def kernel(inputs, emb_table, W, b):
    B, S = inputs.shape
    V, D = emb_table.shape
    idx3 = inputs.astype(jnp.int32).reshape(NW, (B // NW) * 2, HALF)
    pooled = _make_pool_kernel(B, S, V, D)(idx3, emb_table)
    return _matmul_bias(pooled, W, b)
